# baseline (speedup 1.0000x reference)
import functools

import jax
import jax.numpy as jnp
from jax import lax
from jax.experimental import pallas as pl
from jax.experimental.pallas import tpu as pltpu
from jax.experimental.pallas import tpu_sc as plsc


@functools.lru_cache(maxsize=None)
def _build_gather_max(V, D, B, L, CH, NCH, NC, NS):
    NW = NC * NS
    BPW = B // NW
    DV = D // 16
    ILP = 8

    mesh = plsc.VectorSubcoreMesh(core_axis_name="c", subcore_axis_name="s")

    @functools.partial(
        pl.kernel,
        out_type=jax.ShapeDtypeStruct((B, D), jnp.float32),
        mesh=mesh,
        scratch_types=[
            pltpu.VMEM((BPW, NCH, CH), jnp.int32),
            pltpu.VMEM((2, NCH, CH, D), jnp.float32),
            pltpu.VMEM((BPW, D), jnp.float32),
            pltpu.SemaphoreType.DMA,
            pltpu.SemaphoreType.DMA,
        ],
        compiler_params=pltpu.CompilerParams(use_tc_tiling_on_sc=False),
    )
    def gmax(table_hbm, idx_hbm, out_hbm, idx_v, rows_v, pooled_v, sem0, sem1):
        wid = lax.axis_index("s") * NC + lax.axis_index("c")
        base = wid * BPW
        sems = (sem0, sem1)

        pltpu.sync_copy(idx_hbm.at[wid], idx_v)

        def issue(row, b):
            for j in range(NCH):
                pltpu.async_copy(
                    table_hbm.at[idx_v.at[row, j]], rows_v.at[b, j], sems[b])

        def wait(row, b):
            for j in range(NCH):
                pltpu.make_async_copy(
                    table_hbm.at[idx_v.at[row, j]], rows_v.at[b, j], sems[b]).wait()

        def compute(i, b):
            accs = [[None] * DV for _ in range(ILP)]
            for t in range(L):
                j, tt = divmod(t, CH)
                k = t % ILP
                for dv in range(DV):
                    v = rows_v[b, j, tt, pl.ds(dv * 16, 16)]
                    if accs[k][dv] is None:
                        accs[k][dv] = v
                    else:
                        accs[k][dv] = jnp.maximum(accs[k][dv], v)
            while len(accs) > 1:
                nxt = []
                for p in range(0, len(accs) - 1, 2):
                    nxt.append([jnp.maximum(a, c)
                                for a, c in zip(accs[p], accs[p + 1])])
                if len(accs) % 2:
                    nxt.append(accs[-1])
                accs = nxt
            for dv in range(DV):
                pooled_v[i, pl.ds(dv * 16, 16)] = accs[0][dv]

        issue(0, 0)
        issue(1, 1)

        def outer(i0, carry):
            for b in range(2):
                i = 2 * i0 + b
                wait(i, b)
                compute(i, b)

                @pl.when(i + 2 < BPW)
                def _():
                    issue(i + 2, b)
            return carry

        lax.fori_loop(0, BPW // 2, outer, 0)
        pltpu.sync_copy(pooled_v, out_hbm.at[pl.ds(base, BPW)])

    return gmax


@functools.lru_cache(maxsize=None)
def _build_detile(V, D, VB):
    PK = 128 // D

    def body(x_ref, y_ref):
        xt = jnp.transpose(x_ref[...], (1, 0))
        x3 = xt.reshape(VB // PK, PK, D)
        for c in range(PK):
            y_ref[:, D * c:D * (c + 1)] = x3[:, c, :]

    return pl.pallas_call(
        body,
        grid=(pl.cdiv(V, VB),),
        in_specs=[pl.BlockSpec((D, VB), lambda i: (0, i))],
        out_specs=pl.BlockSpec((VB // PK, 128), lambda i: (i, 0)),
        out_shape=jax.ShapeDtypeStruct((V * D // 128, 128), jnp.float32),
    )


@functools.lru_cache(maxsize=None)
def _build_fc(B, D, NL):
    def body(x_ref, w_ref, b_ref, o_ref):
        o_ref[...] = (
            jnp.dot(x_ref[...], w_ref[...], preferred_element_type=jnp.float32)
            + b_ref[...])

    return pl.pallas_call(
        body,
        out_shape=jax.ShapeDtypeStruct((B, NL), jnp.float32),
    )


def kernel(word_ids, table, fc_W, fc_b):
    B, L = word_ids.shape
    V, D = table.shape
    NL = fc_W.shape[0]

    info = plsc.get_sparse_core_info()
    NC, NS = info.num_cores, info.num_subcores
    NW = NC * NS

    NCH = -(-L // 128)
    assert L % NCH == 0, (L, NCH)
    CH = L // NCH
    BPW = B // NW
    assert B % NW == 0 and BPW % 2 == 0 and D % 16 == 0

    idx = word_ids.reshape(NW, BPW, NCH, CH)
    table_lin = _build_detile(V, D, 16384)(table.T).reshape(V, D)
    pooled = _build_gather_max(V, D, B, L, CH, NCH, NC, NS)(table_lin, idx)
    out = _build_fc(B, D, NL)(pooled, fc_W.T, fc_b.reshape(1, NL))
    return out


# speedup vs baseline: 1.0087x; 1.0087x over previous
import functools

import jax
import jax.numpy as jnp
from jax import lax
from jax.experimental import pallas as pl
from jax.experimental.pallas import tpu as pltpu
from jax.experimental.pallas import tpu_sc as plsc


@functools.lru_cache(maxsize=None)
def _build_gather_max(V, D, B, L, CH, NCH, NC, NS):
    NW = NC * NS
    BPW = B // NW
    DV = D // 16
    ILP = 4

    mesh = plsc.VectorSubcoreMesh(core_axis_name="c", subcore_axis_name="s")

    @functools.partial(
        pl.kernel,
        out_type=jax.ShapeDtypeStruct((B, D), jnp.float32),
        mesh=mesh,
        scratch_types=[
            pltpu.VMEM((BPW, NCH, CH), jnp.int32),
            pltpu.VMEM((2, NCH, CH, D), jnp.float32),
            pltpu.VMEM((BPW, D), jnp.float32),
            pltpu.SemaphoreType.DMA,
            pltpu.SemaphoreType.DMA,
        ],
        compiler_params=pltpu.CompilerParams(use_tc_tiling_on_sc=False),
    )
    def gmax(table_hbm, idx_hbm, out_hbm, idx_v, rows_v, pooled_v, sem0, sem1):
        wid = lax.axis_index("s") * NC + lax.axis_index("c")
        base = wid * BPW
        sems = (sem0, sem1)

        pltpu.sync_copy(idx_hbm.at[wid], idx_v)

        def issue(row, b):
            for j in range(NCH):
                pltpu.async_copy(
                    table_hbm.at[idx_v.at[row, j]], rows_v.at[b, j], sems[b])

        def wait(row, b):
            for j in range(NCH):
                pltpu.make_async_copy(
                    table_hbm.at[idx_v.at[row, j]], rows_v.at[b, j], sems[b]).wait()

        def compute(i, b):
            accs = [[None] * DV for _ in range(ILP)]
            for t in range(L):
                j, tt = divmod(t, CH)
                k = t % ILP
                for dv in range(DV):
                    v = rows_v[b, j, tt, pl.ds(dv * 16, 16)]
                    if accs[k][dv] is None:
                        accs[k][dv] = v
                    else:
                        accs[k][dv] = jnp.maximum(accs[k][dv], v)
            while len(accs) > 1:
                nxt = []
                for p in range(0, len(accs) - 1, 2):
                    nxt.append([jnp.maximum(a, c)
                                for a, c in zip(accs[p], accs[p + 1])])
                if len(accs) % 2:
                    nxt.append(accs[-1])
                accs = nxt
            for dv in range(DV):
                pooled_v[i, pl.ds(dv * 16, 16)] = accs[0][dv]

        issue(0, 0)
        issue(1, 1)

        def outer(i0, carry):
            for b in range(2):
                i = 2 * i0 + b
                wait(i, b)
                compute(i, b)

                @pl.when(i + 2 < BPW)
                def _():
                    issue(i + 2, b)
            return carry

        lax.fori_loop(0, BPW // 2, outer, 0)
        pltpu.sync_copy(pooled_v, out_hbm.at[pl.ds(base, BPW)])

    return gmax


@functools.lru_cache(maxsize=None)
def _build_detile(V, D, VB):
    PK = 128 // D

    def body(x_ref, y_ref):
        xt = jnp.transpose(x_ref[...], (1, 0))
        x3 = xt.reshape(VB // PK, PK, D)
        for c in range(PK):
            y_ref[:, D * c:D * (c + 1)] = x3[:, c, :]

    return pl.pallas_call(
        body,
        grid=(pl.cdiv(V, VB),),
        in_specs=[pl.BlockSpec((D, VB), lambda i: (0, i))],
        out_specs=pl.BlockSpec((VB // PK, 128), lambda i: (i, 0)),
        out_shape=jax.ShapeDtypeStruct((V * D // 128, 128), jnp.float32),
    )


@functools.lru_cache(maxsize=None)
def _build_fc(B, D, NL):
    def body(x_ref, w_ref, b_ref, o_ref):
        o_ref[...] = (
            jnp.dot(x_ref[...], w_ref[...], preferred_element_type=jnp.float32)
            + b_ref[...])

    return pl.pallas_call(
        body,
        out_shape=jax.ShapeDtypeStruct((B, NL), jnp.float32),
    )


def kernel(word_ids, table, fc_W, fc_b):
    B, L = word_ids.shape
    V, D = table.shape
    NL = fc_W.shape[0]

    info = plsc.get_sparse_core_info()
    NC, NS = info.num_cores, info.num_subcores
    NW = NC * NS

    NCH = -(-L // 128)
    assert L % NCH == 0, (L, NCH)
    CH = L // NCH
    BPW = B // NW
    assert B % NW == 0 and BPW % 2 == 0 and D % 16 == 0

    idx = word_ids.reshape(NW, BPW, NCH, CH)
    table_lin = _build_detile(V, D, 16384)(table.T).reshape(V, D)
    pooled = _build_gather_max(V, D, B, L, CH, NCH, NC, NS)(table_lin, idx)
    out = _build_fc(B, D, NL)(pooled, fc_W.T, fc_b.reshape(1, NL))
    return out
